# baseline (speedup 1.0000x reference)
import dataclasses
import functools

import jax
import jax.numpy as jnp
from jax import lax
from jax.experimental import pallas as pl
from jax.experimental.pallas import tpu as pltpu
from jax.experimental.pallas import tpu_sc as plsc

_N = 10000
_E = 320000
_F_IN = 128
_H = 8
_D = 32
_HD = _H * _D
_C = _HD // 2
_NSUB = 16
_EB = 80
_EPT = _E // _NSUB
_NBAT = _EPT // _EB
_NCH = 40
_NQ = _N // _NCH
_NQT = (_NQ + _NSUB - 1) // _NSUB


def _tc_proj(feat, W, a16):
    bn = 1000

    def body(feat_ref, w_ref, a_ref, proj_ref, comb_ref):
        proj = jnp.dot(feat_ref[...], w_ref[...],
                       preferred_element_type=jnp.float32)
        proj_ref[...] = proj
        comb_ref[...] = jnp.dot(proj, a_ref[...],
                                preferred_element_type=jnp.float32)

    return pl.pallas_call(
        body,
        grid=(_N // bn,),
        in_specs=[
            pl.BlockSpec((bn, _F_IN), lambda i: (i, 0)),
            pl.BlockSpec((_F_IN, _HD), lambda i: (0, 0)),
            pl.BlockSpec((_HD, 16), lambda i: (0, 0)),
        ],
        out_specs=[
            pl.BlockSpec((bn, _HD), lambda i: (i, 0)),
            pl.BlockSpec((bn, 16), lambda i: (i, 0)),
        ],
        out_shape=[
            jax.ShapeDtypeStruct((_N, _HD), jnp.float32),
            jax.ShapeDtypeStruct((_N, 16), jnp.float32),
        ],
    )(feat, W, a16)


def _tc_edges(edge_index):
    be = 32000

    def body(ei_ref, gs_ref, gd_ref):
        s2 = ei_ref[0:1, :] * 2
        d2 = ei_ref[1:2, :] * 2
        gs_ref[0:1, :] = s2
        gs_ref[1:2, :] = s2 + 1
        gd_ref[0:1, :] = d2
        gd_ref[1:2, :] = d2 + 1

    return pl.pallas_call(
        body,
        grid=(_E // be,),
        in_specs=[pl.BlockSpec((2, be), lambda i: (0, i))],
        out_specs=[
            pl.BlockSpec((2, be), lambda i: (0, i)),
            pl.BlockSpec((2, be), lambda i: (0, i)),
        ],
        out_shape=[
            jax.ShapeDtypeStruct((2, _E), jnp.int32),
            jax.ShapeDtypeStruct((2, _E), jnp.int32),
        ],
    )(edge_index)


def _sc_gat(proj2, gsrc2, gdst2, edge_index, combi, bias2):
    mesh = plsc.VectorSubcoreMesh(core_axis_name="c", subcore_axis_name="s")
    cp = pltpu.CompilerParams()
    if "needs_layout_passes" in pltpu.CompilerParams.__dataclass_fields__:
        cp = dataclasses.replace(cp, needs_layout_passes=False)
    if "use_tc_tiling_on_sc" in pltpu.CompilerParams.__dataclass_fields__:
        cp = dataclasses.replace(cp, use_tc_tiling_on_sc=False)

    @functools.partial(
        pl.kernel,
        out_type=jax.ShapeDtypeStruct((2, _N, _C), jnp.float32),
        mesh=mesh,
        compiler_params=cp,
        scratch_types=[
            pltpu.VMEM((_C,), jnp.float32),
            pltpu.VMEM((3, _EB), jnp.int32),
            pltpu.VMEM((3, _EB), jnp.int32),
            pltpu.VMEM((3, _EB), jnp.int32),
            pltpu.VMEM((3, _EB, 8), jnp.float32),
            pltpu.VMEM((3, _EB, 8), jnp.float32),
            pltpu.VMEM((3, _EB, _C), jnp.float32),
            pltpu.VMEM((3, _EB, 16), jnp.float32),
            pltpu.VMEM_SHARED((_N, _C), jnp.float32),
            pltpu.VMEM_SHARED((_N, 16), jnp.float32),
            pltpu.SemaphoreType.DMA,
            pltpu.SemaphoreType.DMA,
            pltpu.SemaphoreType.DMA,
            pltpu.SemaphoreType.DMA,
            pltpu.SemaphoreType.DMA,
            pltpu.SemaphoreType.DMA,
            pltpu.SemaphoreType.DMA,
            pltpu.SemaphoreType.DMA,
            pltpu.SemaphoreType.DMA,
        ],
    )
    def k(proj_hbm, gsrc_hbm, gdst_hbm, edge_hbm, combi_hbm, bias_hbm,
          out_hbm, bias_v, gidx_r, didx_r, dstb_r, cs_r, cd_r, rows_r,
          wbuf_r, acc_sh, den_sh,
          semi0, semi1, semi2, semg0, semg1, semg2, sems0, sems1, sems2):
        semi = [semi0, semi1, semi2]
        semg = [semg0, semg1, semg2]
        sems = [sems0, sems1, sems2]
        c = lax.axis_index("c")
        s = lax.axis_index("s")

        lane = lax.iota(jnp.int32, 16)
        ebase = s * _EPT

        def idx_start(b, p):
            off = ebase + b * _EB
            pltpu.async_copy(gsrc_hbm.at[c, pl.ds(off, _EB)],
                             gidx_r.at[p], semi[p])
            pltpu.async_copy(gdst_hbm.at[c, pl.ds(off, _EB)],
                             didx_r.at[p], semi[p])

        def idx_wait(b, p):
            off = ebase + b * _EB
            pltpu.make_async_copy(gsrc_hbm.at[c, pl.ds(off, _EB)],
                                  gidx_r.at[p], semi[p]).wait()
            pltpu.make_async_copy(gdst_hbm.at[c, pl.ds(off, _EB)],
                                  didx_r.at[p], semi[p]).wait()

        def gather_start(b, p):
            off = ebase + b * _EB
            pltpu.async_copy(edge_hbm.at[1, pl.ds(off, _EB)],
                             dstb_r.at[p], semg[p])
            pltpu.async_copy(combi_hbm.at[gidx_r.at[p]], cs_r.at[p],
                             semg[p])
            pltpu.async_copy(combi_hbm.at[didx_r.at[p]], cd_r.at[p],
                             semg[p])
            pltpu.async_copy(proj_hbm.at[gidx_r.at[p]], rows_r.at[p],
                             semg[p])

        def gather_wait(b, p):
            off = ebase + b * _EB
            pltpu.make_async_copy(edge_hbm.at[1, pl.ds(off, _EB)],
                                  dstb_r.at[p], semg[p]).wait()
            pltpu.make_async_copy(combi_hbm.at[gidx_r.at[p]], cs_r.at[p],
                                  semg[p]).wait()
            pltpu.make_async_copy(combi_hbm.at[didx_r.at[p]], cd_r.at[p],
                                  semg[p]).wait()
            pltpu.make_async_copy(proj_hbm.at[gidx_r.at[p]], rows_r.at[p],
                                  semg[p]).wait()

        def scatter_start(p):
            pltpu.async_copy(rows_r.at[p], acc_sh.at[dstb_r.at[p]],
                             sems[p], add=True)
            pltpu.async_copy(wbuf_r.at[p], den_sh.at[dstb_r.at[p]],
                             sems[p], add=True)

        def scatter_wait(p):
            pltpu.make_async_copy(rows_r.at[p], acc_sh.at[dstb_r.at[p]],
                                  sems[p]).wait()
            pltpu.make_async_copy(wbuf_r.at[p], den_sh.at[dstb_r.at[p]],
                                  sems[p]).wait()

        idx_start(0, 0)
        idx_start(1, 1)
        pltpu.sync_copy(bias_hbm.at[pl.multiple_of(c * 8, 8)], bias_v)

        zeros16 = jnp.zeros((16,), jnp.float32)

        @pl.loop(0, _EB)
        def _(r):
            for k8 in range(_C // 16):
                rows_r[2, r, pl.ds(16 * k8, 16)] = zeros16
            for p in range(3):
                wbuf_r[p, r, pl.ds(0, 16)] = zeros16

        idx_wait(0, 0)
        gather_start(0, 0)

        @pl.loop(0, _NQT)
        def _(t):
            q = s + _NSUB * t

            @pl.when(q < _NQ)
            def _():
                nb = pl.multiple_of(q * _NCH, 8)
                pltpu.async_copy(rows_r.at[2, pl.ds(0, _NCH)],
                                 acc_sh.at[pl.ds(nb, _NCH)], sems[2])
                pltpu.async_copy(wbuf_r.at[2, pl.ds(0, _NCH)],
                                 den_sh.at[pl.ds(nb, _NCH)], sems[2])

        @pl.loop(0, _NQT)
        def _(t):
            q = s + _NSUB * t

            @pl.when(q < _NQ)
            def _():
                nb = pl.multiple_of(q * _NCH, 8)
                pltpu.make_async_copy(rows_r.at[2, pl.ds(0, _NCH)],
                                      acc_sh.at[pl.ds(nb, _NCH)],
                                      sems[2]).wait()
                pltpu.make_async_copy(wbuf_r.at[2, pl.ds(0, _NCH)],
                                      den_sh.at[pl.ds(nb, _NCH)],
                                      sems[2]).wait()

        plsc.subcore_barrier()

        @pl.loop(0, _NBAT + 2, step=3)
        def _(i0):
            for u in range(3):
                i = i0 + u

                @pl.when(i < _NBAT)
                def _():
                    p = u
                    p1 = (u + 1) % 3
                    p2 = (u + 2) % 3

                    @pl.when(i + 2 < _NBAT)
                    def _():
                        idx_start(i + 2, p2)

                    @pl.when(i >= 2)
                    def _():
                        scatter_wait(p1)

                    @pl.when(i + 1 < _NBAT)
                    def _():
                        idx_wait(i + 1, p1)
                        gather_start(i + 1, p1)

                    gather_wait(i, p)

                    @plsc.parallel_loop(0, _EB, step=16, unroll=5)
                    def _(eb):
                        e16 = lane + eb
                        for h in range(4):
                            el = plsc.load_gather(
                                cs_r.at[p], [e16, jnp.full((16,), h, jnp.int32)])
                            er = plsc.load_gather(
                                cd_r.at[p], [e16, jnp.full((16,), h + 4, jnp.int32)])
                            x = el + er
                            x = jnp.maximum(x, 0.2 * x)
                            w = jnp.exp(x)
                            plsc.store_scatter(
                                wbuf_r.at[p], [e16, jnp.full((16,), h, jnp.int32)], w)

                    @plsc.parallel_loop(0, _EB, step=1, unroll=16)
                    def _(e):
                        w16 = wbuf_r[p, e, pl.ds(0, 16)]
                        for h in range(4):
                            w = w16[h]
                            for kk in range(2):
                                sl = pl.ds(32 * h + 16 * kk, 16)
                                rows_r[p, e, sl] = rows_r[p, e, sl] * w

                    scatter_start(p)

        scatter_wait((_NBAT - 2) % 3)
        scatter_wait((_NBAT - 1) % 3)

        plsc.subcore_barrier()

        def fnb(t):
            return pl.multiple_of((s + _NSUB * t) * _NCH, 8)

        def fvalid(t):
            return s + _NSUB * t < _NQ

        def fin_in_start(t, u):
            nb = fnb(t)
            pltpu.async_copy(acc_sh.at[pl.ds(nb, _NCH)],
                             rows_r.at[u, pl.ds(0, _NCH)], semi[u])
            pltpu.async_copy(den_sh.at[pl.ds(nb, _NCH)],
                             wbuf_r.at[u, pl.ds(0, _NCH)], semi[u])

        def fin_in_wait(t, u):
            nb = fnb(t)
            pltpu.make_async_copy(acc_sh.at[pl.ds(nb, _NCH)],
                                  rows_r.at[u, pl.ds(0, _NCH)],
                                  semi[u]).wait()
            pltpu.make_async_copy(den_sh.at[pl.ds(nb, _NCH)],
                                  wbuf_r.at[u, pl.ds(0, _NCH)],
                                  semi[u]).wait()

        def fin_out_start(t, u):
            nb = fnb(t)
            pltpu.async_copy(rows_r.at[u, pl.ds(0, _NCH)],
                             out_hbm.at[c, pl.ds(nb, _NCH)], semg[u])

        def fin_out_wait(t, u):
            nb = fnb(t)
            pltpu.make_async_copy(rows_r.at[u, pl.ds(0, _NCH)],
                                  out_hbm.at[c, pl.ds(nb, _NCH)],
                                  semg[u]).wait()

        fin_in_start(0, 0)

        @pl.loop(0, _NQT, step=2)
        def _(t0):
            for u in range(2):
                t = t0 + u

                @pl.when(fvalid(t))
                def _():
                    @pl.when(t >= 1)
                    def _():
                        fin_out_wait(t - 1, (u + 1) % 2)

                    @pl.when(fvalid(t + 1))
                    def _():
                        fin_in_start(t + 1, (u + 1) % 2)

                    fin_in_wait(t, u)

                    @plsc.parallel_loop(0, _NCH, step=1, unroll=4)
                    def _(r):
                        d16 = jnp.maximum(wbuf_r[u, r, pl.ds(0, 16)],
                                          jnp.float32(1e-30))
                        for h in range(4):
                            d = d16[h]
                            for kk in range(2):
                                sl = pl.ds(32 * h + 16 * kk, 16)
                                rows_r[u, r, sl] = (
                                    rows_r[u, r, sl] / d + bias_v[sl])

                    fin_out_start(t, u)

        last_valid = fvalid(_NQT - 1)

        @pl.when(last_valid)
        def _():
            fin_out_wait(_NQT - 1, (_NQT - 1) % 2)

        @pl.when(jnp.logical_not(last_valid))
        def _():
            fin_out_wait(_NQT - 2, (_NQT - 2) % 2)

    return k(proj2, gsrc2, gdst2, edge_index, combi, bias2)


def kernel(feat, edge_index, W, attn_l, attn_r, bias):
    eye = jnp.eye(_H, dtype=jnp.float32)
    al = (attn_l[0][:, :, None] * eye[:, None, :]).reshape(_HD, _H)
    ar = (attn_r[0][:, :, None] * eye[:, None, :]).reshape(_HD, _H)
    a16 = jnp.concatenate(
        [al[:, 0:4], ar[:, 0:4], al[:, 4:8], ar[:, 4:8]], axis=1)

    proj, comb16 = _tc_proj(feat, W, a16)
    proj2 = proj.reshape(2 * _N, _C)
    combi = comb16.reshape(2 * _N, 8)
    bias2 = jnp.zeros((2, 8, _C), jnp.float32).at[:, 0, :].set(
        bias.reshape(2, _C)).reshape(16, _C)

    gsrc2, gdst2 = _tc_edges(edge_index)

    out = _sc_gat(proj2, gsrc2, gdst2, edge_index, combi, bias2)
    return out.transpose(1, 0, 2).reshape(_N, _H, _D)


# speedup vs baseline: 1.1336x; 1.1336x over previous
import dataclasses
import functools

import jax
import jax.numpy as jnp
from jax import lax
from jax.experimental import pallas as pl
from jax.experimental.pallas import tpu as pltpu
from jax.experimental.pallas import tpu_sc as plsc

_N = 10000
_E = 320000
_F_IN = 128
_H = 8
_D = 32
_HD = _H * _D
_C = _HD // 2
_NSUB = 16
_EB = 80
_EPT = _E // _NSUB
_NBAT = _EPT // _EB
_NCH = 40
_NQ = _N // _NCH
_NQT = (_NQ + _NSUB - 1) // _NSUB


def _tc_proj(feat, W, a16):
    bn = 1000

    def body(feat_ref, w_ref, a_ref, proj_ref, comb_ref):
        proj = jnp.dot(feat_ref[...], w_ref[...],
                       preferred_element_type=jnp.float32)
        proj_ref[...] = proj
        comb_ref[...] = jnp.dot(proj, a_ref[...],
                                preferred_element_type=jnp.float32)

    return pl.pallas_call(
        body,
        grid=(_N // bn,),
        in_specs=[
            pl.BlockSpec((bn, _F_IN), lambda i: (i, 0)),
            pl.BlockSpec((_F_IN, _HD), lambda i: (0, 0)),
            pl.BlockSpec((_HD, 16), lambda i: (0, 0)),
        ],
        out_specs=[
            pl.BlockSpec((bn, _HD), lambda i: (i, 0)),
            pl.BlockSpec((bn, 16), lambda i: (i, 0)),
        ],
        out_shape=[
            jax.ShapeDtypeStruct((_N, _HD), jnp.float32),
            jax.ShapeDtypeStruct((_N, 16), jnp.float32),
        ],
    )(feat, W, a16)


def _tc_edges(edge_index):
    be = 32000

    def body(ei_ref, gs_ref, gd_ref):
        s2 = ei_ref[0:1, :] * 2
        d2 = ei_ref[1:2, :] * 2
        gs_ref[0:1, :] = s2
        gs_ref[1:2, :] = s2 + 1
        gd_ref[0:1, :] = d2
        gd_ref[1:2, :] = d2 + 1

    return pl.pallas_call(
        body,
        grid=(_E // be,),
        in_specs=[pl.BlockSpec((2, be), lambda i: (0, i))],
        out_specs=[
            pl.BlockSpec((2, be), lambda i: (0, i)),
            pl.BlockSpec((2, be), lambda i: (0, i)),
        ],
        out_shape=[
            jax.ShapeDtypeStruct((2, _E), jnp.int32),
            jax.ShapeDtypeStruct((2, _E), jnp.int32),
        ],
    )(edge_index)


def _sc_gat(proj2, gsrc2, gdst2, edge_index, combi, bias2):
    mesh = plsc.VectorSubcoreMesh(core_axis_name="c", subcore_axis_name="s")
    cp = pltpu.CompilerParams()
    if "needs_layout_passes" in pltpu.CompilerParams.__dataclass_fields__:
        cp = dataclasses.replace(cp, needs_layout_passes=False)
    if "use_tc_tiling_on_sc" in pltpu.CompilerParams.__dataclass_fields__:
        cp = dataclasses.replace(cp, use_tc_tiling_on_sc=False)

    @functools.partial(
        pl.kernel,
        out_type=jax.ShapeDtypeStruct((2, _N, _C), jnp.float32),
        mesh=mesh,
        compiler_params=cp,
        scratch_types=[
            pltpu.VMEM((_C,), jnp.float32),
            pltpu.VMEM((3, _EB), jnp.int32),
            pltpu.VMEM((3, _EB), jnp.int32),
            pltpu.VMEM((3, _EB), jnp.int32),
            pltpu.VMEM((3, _EB, 8), jnp.float32),
            pltpu.VMEM((3, _EB, 8), jnp.float32),
            pltpu.VMEM((3, _EB, _C), jnp.float32),
            pltpu.VMEM((3, _EB, 16), jnp.float32),
            pltpu.VMEM_SHARED((_N, _C), jnp.float32),
            pltpu.VMEM_SHARED((_N, 16), jnp.float32),
            pltpu.SemaphoreType.DMA,
            pltpu.SemaphoreType.DMA,
            pltpu.SemaphoreType.DMA,
            pltpu.SemaphoreType.DMA,
            pltpu.SemaphoreType.DMA,
            pltpu.SemaphoreType.DMA,
            pltpu.SemaphoreType.DMA,
            pltpu.SemaphoreType.DMA,
            pltpu.SemaphoreType.DMA,
        ],
    )
    def k(proj_hbm, gsrc_hbm, gdst_hbm, edge_hbm, combi_hbm, bias_hbm,
          out_hbm, bias_v, gidx_r, didx_r, dstb_r, cs_r, cd_r, rows_r,
          wbuf_r, acc_sh, den_sh,
          semi0, semi1, semi2, semg0, semg1, semg2, sems0, sems1, sems2):
        semi = [semi0, semi1, semi2]
        semg = [semg0, semg1, semg2]
        sems = [sems0, sems1, sems2]
        c = lax.axis_index("c")
        s = lax.axis_index("s")

        lane = lax.iota(jnp.int32, 16)
        ebase = s * _EPT

        def idx_start(b, p):
            off = ebase + b * _EB
            pltpu.async_copy(gsrc_hbm.at[c, pl.ds(off, _EB)],
                             gidx_r.at[p], semi[p])
            pltpu.async_copy(gdst_hbm.at[c, pl.ds(off, _EB)],
                             didx_r.at[p], semi[p])

        def idx_wait(b, p):
            off = ebase + b * _EB
            pltpu.make_async_copy(gsrc_hbm.at[c, pl.ds(off, _EB)],
                                  gidx_r.at[p], semi[p]).wait()
            pltpu.make_async_copy(gdst_hbm.at[c, pl.ds(off, _EB)],
                                  didx_r.at[p], semi[p]).wait()

        def gather_start(b, p):
            off = ebase + b * _EB
            pltpu.async_copy(edge_hbm.at[1, pl.ds(off, _EB)],
                             dstb_r.at[p], semg[p])
            pltpu.async_copy(combi_hbm.at[gidx_r.at[p]], cs_r.at[p],
                             semg[p])
            pltpu.async_copy(combi_hbm.at[didx_r.at[p]], cd_r.at[p],
                             semg[p])
            pltpu.async_copy(proj_hbm.at[gidx_r.at[p]], rows_r.at[p],
                             semg[p])

        def gather_wait(b, p):
            off = ebase + b * _EB
            pltpu.make_async_copy(edge_hbm.at[1, pl.ds(off, _EB)],
                                  dstb_r.at[p], semg[p]).wait()
            pltpu.make_async_copy(combi_hbm.at[gidx_r.at[p]], cs_r.at[p],
                                  semg[p]).wait()
            pltpu.make_async_copy(combi_hbm.at[didx_r.at[p]], cd_r.at[p],
                                  semg[p]).wait()
            pltpu.make_async_copy(proj_hbm.at[gidx_r.at[p]], rows_r.at[p],
                                  semg[p]).wait()

        def scatter_start(p):
            pltpu.async_copy(rows_r.at[p], acc_sh.at[dstb_r.at[p]],
                             sems[p], add=True)
            pltpu.async_copy(wbuf_r.at[p], den_sh.at[dstb_r.at[p]],
                             sems[p], add=True)

        def scatter_wait(p):
            pltpu.make_async_copy(rows_r.at[p], acc_sh.at[dstb_r.at[p]],
                                  sems[p]).wait()
            pltpu.make_async_copy(wbuf_r.at[p], den_sh.at[dstb_r.at[p]],
                                  sems[p]).wait()

        idx_start(0, 0)
        idx_start(1, 1)
        pltpu.sync_copy(bias_hbm.at[pl.multiple_of(c * 8, 8)], bias_v)

        zeros16 = jnp.zeros((16,), jnp.float32)

        @pl.loop(0, _EB)
        def _(r):
            for k8 in range(_C // 16):
                rows_r[2, r, pl.ds(16 * k8, 16)] = zeros16
            for p in range(3):
                wbuf_r[p, r, pl.ds(0, 16)] = zeros16

        idx_wait(0, 0)
        gather_start(0, 0)

        @pl.loop(0, _NQT)
        def _(t):
            q = s + _NSUB * t

            @pl.when(q < _NQ)
            def _():
                nb = pl.multiple_of(q * _NCH, 8)
                pltpu.async_copy(rows_r.at[2, pl.ds(0, _NCH)],
                                 acc_sh.at[pl.ds(nb, _NCH)], sems[2])
                pltpu.async_copy(wbuf_r.at[2, pl.ds(0, _NCH)],
                                 den_sh.at[pl.ds(nb, _NCH)], sems[2])

        @pl.loop(0, _NQT)
        def _(t):
            q = s + _NSUB * t

            @pl.when(q < _NQ)
            def _():
                nb = pl.multiple_of(q * _NCH, 8)
                pltpu.make_async_copy(rows_r.at[2, pl.ds(0, _NCH)],
                                      acc_sh.at[pl.ds(nb, _NCH)],
                                      sems[2]).wait()
                pltpu.make_async_copy(wbuf_r.at[2, pl.ds(0, _NCH)],
                                      den_sh.at[pl.ds(nb, _NCH)],
                                      sems[2]).wait()

        plsc.subcore_barrier()

        @pl.loop(0, _NBAT + 2, step=3)
        def _(i0):
            for u in range(3):
                i = i0 + u

                @pl.when(i < _NBAT)
                def _():
                    p = u
                    p1 = (u + 1) % 3
                    p2 = (u + 2) % 3

                    @pl.when(i + 2 < _NBAT)
                    def _():
                        idx_start(i + 2, p2)

                    @pl.when(i >= 2)
                    def _():
                        scatter_wait(p1)

                    @pl.when(i + 1 < _NBAT)
                    def _():
                        idx_wait(i + 1, p1)
                        gather_start(i + 1, p1)

                    gather_wait(i, p)

                    @plsc.parallel_loop(0, _EB, step=16, unroll=5)
                    def _(eb):
                        e16 = lane + eb
                        for h in range(4):
                            el = plsc.load_gather(
                                cs_r.at[p], [e16, jnp.full((16,), h, jnp.int32)])
                            er = plsc.load_gather(
                                cd_r.at[p], [e16, jnp.full((16,), h + 4, jnp.int32)])
                            x = el + er
                            x = jnp.maximum(x, 0.2 * x)
                            w = jnp.exp(x)
                            plsc.store_scatter(
                                wbuf_r.at[p], [e16, jnp.full((16,), h, jnp.int32)], w)

                    @plsc.parallel_loop(0, _EB, step=1, unroll=8)
                    def _(e):
                        w16 = wbuf_r[p, e, pl.ds(0, 16)]
                        for h in range(4):
                            w = w16[h]
                            for kk in range(2):
                                sl = pl.ds(32 * h + 16 * kk, 16)
                                rows_r[p, e, sl] = rows_r[p, e, sl] * w

                    scatter_start(p)

        scatter_wait((_NBAT - 2) % 3)
        scatter_wait((_NBAT - 1) % 3)

        plsc.subcore_barrier()

        def fnb(t):
            return pl.multiple_of((s + _NSUB * t) * _NCH, 8)

        def fvalid(t):
            return s + _NSUB * t < _NQ

        def fin_in_start(t, u):
            nb = fnb(t)
            pltpu.async_copy(acc_sh.at[pl.ds(nb, _NCH)],
                             rows_r.at[u, pl.ds(0, _NCH)], semi[u])
            pltpu.async_copy(den_sh.at[pl.ds(nb, _NCH)],
                             wbuf_r.at[u, pl.ds(0, _NCH)], semi[u])

        def fin_in_wait(t, u):
            nb = fnb(t)
            pltpu.make_async_copy(acc_sh.at[pl.ds(nb, _NCH)],
                                  rows_r.at[u, pl.ds(0, _NCH)],
                                  semi[u]).wait()
            pltpu.make_async_copy(den_sh.at[pl.ds(nb, _NCH)],
                                  wbuf_r.at[u, pl.ds(0, _NCH)],
                                  semi[u]).wait()

        def fin_out_start(t, u):
            nb = fnb(t)
            pltpu.async_copy(rows_r.at[u, pl.ds(0, _NCH)],
                             out_hbm.at[c, pl.ds(nb, _NCH)], semg[u])

        def fin_out_wait(t, u):
            nb = fnb(t)
            pltpu.make_async_copy(rows_r.at[u, pl.ds(0, _NCH)],
                                  out_hbm.at[c, pl.ds(nb, _NCH)],
                                  semg[u]).wait()

        fin_in_start(0, 0)

        @pl.loop(0, _NQT, step=2)
        def _(t0):
            for u in range(2):
                t = t0 + u

                @pl.when(fvalid(t))
                def _():
                    @pl.when(t >= 1)
                    def _():
                        fin_out_wait(t - 1, (u + 1) % 2)

                    @pl.when(fvalid(t + 1))
                    def _():
                        fin_in_start(t + 1, (u + 1) % 2)

                    fin_in_wait(t, u)

                    @plsc.parallel_loop(0, _NCH, step=1, unroll=4)
                    def _(r):
                        d16 = jnp.maximum(wbuf_r[u, r, pl.ds(0, 16)],
                                          jnp.float32(1e-30))
                        for h in range(4):
                            d = d16[h]
                            for kk in range(2):
                                sl = pl.ds(32 * h + 16 * kk, 16)
                                rows_r[u, r, sl] = (
                                    rows_r[u, r, sl] / d + bias_v[sl])

                    fin_out_start(t, u)

        last_valid = fvalid(_NQT - 1)

        @pl.when(last_valid)
        def _():
            fin_out_wait(_NQT - 1, (_NQT - 1) % 2)

        @pl.when(jnp.logical_not(last_valid))
        def _():
            fin_out_wait(_NQT - 2, (_NQT - 2) % 2)

    return k(proj2, gsrc2, gdst2, edge_index, combi, bias2)


def kernel(feat, edge_index, W, attn_l, attn_r, bias):
    eye = jnp.eye(_H, dtype=jnp.float32)
    al = (attn_l[0][:, :, None] * eye[:, None, :]).reshape(_HD, _H)
    ar = (attn_r[0][:, :, None] * eye[:, None, :]).reshape(_HD, _H)
    a16 = jnp.concatenate(
        [al[:, 0:4], ar[:, 0:4], al[:, 4:8], ar[:, 4:8]], axis=1)

    proj, comb16 = _tc_proj(feat, W, a16)
    proj2 = proj.reshape(2 * _N, _C)
    combi = comb16.reshape(2 * _N, 8)
    bias2 = jnp.zeros((2, 8, _C), jnp.float32).at[:, 0, :].set(
        bias.reshape(2, _C)).reshape(16, _C)

    gsrc2, gdst2 = _tc_edges(edge_index)

    out = _sc_gat(proj2, gsrc2, gdst2, edge_index, combi, bias2)
    return out.transpose(1, 0, 2).reshape(_N, _H, _D)
